# baseline (speedup 1.0000x reference)
import functools
import math

import jax
import jax.numpy as jnp
from jax.experimental import pallas as pl
from jax.experimental.pallas import tpu as pltpu

_E = 8
_H = 128
_F = 256
_O = 64
_D = _E * _H
_TILE = 512
_SQRT2 = math.sqrt(2.0)


def _moe_body(x_ref, wr_ref, br_ref, gamma_ref, beta_ref, w1_ref, b1_ref,
              w2_ref, b2_ref, o_ref):
    xt = x_ref[:, :]
    logits = jnp.dot(xt, wr_ref[:, :],
                     preferred_element_type=jnp.float32) + br_ref[0, :]
    mu = jnp.mean(logits, axis=-1, keepdims=True)
    var = jnp.mean((logits - mu) ** 2, axis=-1, keepdims=True)
    normed = ((logits - mu) / jnp.sqrt(var + 1e-5)) * gamma_ref[0, :] \
        + beta_ref[0, :]
    m = jnp.max(normed, axis=-1, keepdims=True)
    ex = jnp.exp(normed - m)
    router = ex / jnp.sum(ex, axis=-1, keepdims=True)

    xb = xt.astype(jnp.bfloat16)
    cols = []
    for e in range(_E):
        he = jnp.dot(xb[:, e * _H:(e + 1) * _H], w1_ref[e],
                     preferred_element_type=jnp.float32) + b1_ref[e]
        ge = he * (0.5 + 0.5 * jax.lax.erf(he * (1.0 / _SQRT2)))
        cols.append((router[:, e:e + 1] * ge).astype(jnp.bfloat16))
    gmat = jnp.concatenate(cols, axis=1)

    y = jnp.dot(gmat, w2_ref[:, :], preferred_element_type=jnp.float32)
    y = y + jnp.dot(router, b2_ref[:, :],
                    preferred_element_type=jnp.float32)
    o_ref[:, :] = y


@functools.partial(jax.jit, static_argnames=())
def kernel(x, Wr, br, gamma, beta, W1, b1, W2, b2):
    B, T, D = x.shape
    BT = B * T
    xf = x.reshape(BT, D)
    w1b = W1.astype(jnp.bfloat16)
    w2f = W2.reshape(_E * _F, _O).astype(jnp.bfloat16)
    grid = (BT // _TILE,)

    out = pl.pallas_call(
        _moe_body,
        grid=grid,
        in_specs=[
            pl.BlockSpec((_TILE, D), lambda i: (i, 0)),
            pl.BlockSpec((D, _E), lambda i: (0, 0)),
            pl.BlockSpec((1, _E), lambda i: (0, 0)),
            pl.BlockSpec((1, _E), lambda i: (0, 0)),
            pl.BlockSpec((1, _E), lambda i: (0, 0)),
            pl.BlockSpec((_E, _H, _F), lambda i: (0, 0, 0)),
            pl.BlockSpec((_E, _F), lambda i: (0, 0)),
            pl.BlockSpec((_E * _F, _O), lambda i: (0, 0)),
            pl.BlockSpec((_E, _O), lambda i: (0, 0)),
        ],
        out_specs=pl.BlockSpec((_TILE, _O), lambda i: (i, 0)),
        out_shape=jax.ShapeDtypeStruct((BT, _O), jnp.float32),
        compiler_params=pltpu.CompilerParams(
            dimension_semantics=("parallel",),
        ),
    )(xf, Wr, br.reshape(1, _E), gamma.reshape(1, _E), beta.reshape(1, _E),
      w1b, b1, w2f, b2)
    return out.reshape(B, T, _O)


# speedup vs baseline: 1.1180x; 1.1180x over previous
import math

import jax
import jax.numpy as jnp
from jax.experimental import pallas as pl
from jax.experimental.pallas import tpu as pltpu

_E = 8
_H = 128
_F = 256
_O = 64
_D = _E * _H
_TILE = 1024
_C = math.sqrt(2.0) / 2.0


def _moe_body(x_ref, wr_ref, br_ref, gamma_ref, beta_ref, w1_ref, b1_ref,
              w2_ref, b2_ref, o_ref):
    xt = x_ref[:, :]
    j8 = jnp.full((_E, _E), 1.0 / _E, dtype=jnp.float32)
    ones8 = jnp.ones((_E, _E), dtype=jnp.float32)

    logits = jnp.dot(xt, wr_ref[:, :],
                     preferred_element_type=jnp.float32) + br_ref[0, :]
    mu = jnp.dot(logits, j8, preferred_element_type=jnp.float32)
    d = logits - mu
    var = jnp.dot(d * d, j8, preferred_element_type=jnp.float32)
    normed = d * jax.lax.rsqrt(var + 1e-5) * gamma_ref[0, :] + beta_ref[0, :]
    ex = jnp.exp(normed)
    denom = jnp.dot(ex, ones8, preferred_element_type=jnp.float32)
    router = ex / denom

    xb = xt.astype(jnp.bfloat16)
    acc = jnp.dot(router, b2_ref[:, :], preferred_element_type=jnp.float32)
    for e in range(_E):
        s = jnp.dot(xb[:, e * _H:(e + 1) * _H], w1_ref[e],
                    preferred_element_type=jnp.float32) + b1_ref[e]
        u = _C * s
        ge = u + u * jax.lax.erf(s)
        pe = jnp.dot(ge.astype(jnp.bfloat16), w2_ref[e],
                     preferred_element_type=jnp.float32)
        acc = acc + router[:, e:e + 1] * pe
    o_ref[:, :] = acc


def kernel(x, Wr, br, gamma, beta, W1, b1, W2, b2):
    B, T, D = x.shape
    BT = B * T
    xf = x.reshape(BT, D)
    w1s = (W1 * _C).astype(jnp.bfloat16)
    b1s = b1 * _C
    w2b = W2.astype(jnp.bfloat16)
    grid = (BT // _TILE,)

    out = pl.pallas_call(
        _moe_body,
        grid=grid,
        in_specs=[
            pl.BlockSpec((_TILE, D), lambda i: (i, 0)),
            pl.BlockSpec((D, _E), lambda i: (0, 0)),
            pl.BlockSpec((1, _E), lambda i: (0, 0)),
            pl.BlockSpec((1, _E), lambda i: (0, 0)),
            pl.BlockSpec((1, _E), lambda i: (0, 0)),
            pl.BlockSpec((_E, _H, _F), lambda i: (0, 0, 0)),
            pl.BlockSpec((_E, _F), lambda i: (0, 0)),
            pl.BlockSpec((_E, _F, _O), lambda i: (0, 0, 0)),
            pl.BlockSpec((_E, _O), lambda i: (0, 0)),
        ],
        out_specs=pl.BlockSpec((_TILE, _O), lambda i: (i, 0)),
        out_shape=jax.ShapeDtypeStruct((BT, _O), jnp.float32),
        compiler_params=pltpu.CompilerParams(
            dimension_semantics=("parallel",),
        ),
    )(xf, Wr, br.reshape(1, _E), gamma.reshape(1, _E), beta.reshape(1, _E),
      w1s, b1s, w2b, b2)
    return out.reshape(B, T, _O)
